# baseline (speedup 1.0000x reference)
import numpy as np

import jax
import jax.numpy as jnp
from jax import lax
from jax.experimental import pallas as pl
from jax.experimental.pallas import tpu as pltpu
from jax.experimental.pallas import tpu_sc as plsc

GRID_NUM = 7.0
LAMBDA_COORD = 5.0
LAMBDA_NOOBJ = 0.5

BATCH = 2048
CELLS = 7 * 7
COLS = 26
N_ROWS = BATCH * CELLS
NC, NS = 2, 16
NW = NC * NS
ROWS_PER_W = N_ROWS // NW
CHUNK_ROWS = 784
N_CHUNKS = ROWS_PER_W // CHUNK_ROWS
GROUPS = CHUNK_ROWS // 16
CHUNK_W = CHUNK_ROWS * COLS

_RSQRT_MAGIC = np.int32(0x5F3759DF)


def _sqrt16(x):
    xc = jnp.maximum(x, jnp.float32(1e-12))
    i = lax.bitcast_convert_type(xc, jnp.int32)
    i = _RSQRT_MAGIC - lax.shift_right_arithmetic(i, 1)
    y = lax.bitcast_convert_type(i, jnp.float32)
    half = jnp.float32(0.5) * xc
    for _ in range(2):
        y = y * (jnp.float32(1.5) - half * y * y)
    return xc * y


def _corners(cx, cy, w, h):
    gx = cx * jnp.float32(1.0 / GRID_NUM)
    gy = cy * jnp.float32(1.0 / GRID_NUM)
    hw = jnp.float32(0.5) * w
    hh = jnp.float32(0.5) * h
    return gx - hw, gy - hh, gx + hw, gy + hh


def _iou16(p, t):
    px0, py0, px1, py1 = p
    tx0, ty0, tx1, ty1 = t
    ltx = jnp.maximum(px0, tx0)
    lty = jnp.maximum(py0, ty0)
    rbx = jnp.minimum(px1, tx1)
    rby = jnp.minimum(py1, ty1)
    iw = jnp.maximum(rbx - ltx, jnp.float32(0.0))
    ih = jnp.maximum(rby - lty, jnp.float32(0.0))
    return iw * ih


def _loss_body(o_hbm, t_hbm, out_hbm, ob0, tb0, ob1, tb1, vstage,
               sem0, sem1):
    c = lax.axis_index("c")
    s = lax.axis_index("s")
    wid = s * NC + c
    lane = lax.iota(jnp.int32, 16)
    col0 = lane * COLS
    bufs = ((ob0, tb0), (ob1, tb1))
    sems = (sem0, sem1)

    def make_group_body(ob, tb):
      obc = [ob.at[pl.ds(j & ~7, CHUNK_W - (j & ~7))] for j in range(COLS)]
      tbc = [tb.at[pl.ds(j & ~7, CHUNK_W - (j & ~7))] for j in range(COLS)]

      def group_body(g, acc):
        base = col0 + g * (16 * COLS)
        base_r = [base + r for r in range(8)]

        def go(col):
            return plsc.load_gather(obc[col], [base_r[col & 7]])

        def gt(col):
            return plsc.load_gather(tbc[col], [base_r[col & 7]])

        cls = jnp.zeros((16,), jnp.float32)
        for j in range(10, 26):
            d = go(j) - gt(j)
            cls = cls + d * d
        t4 = gt(4)

        def box_term(off):
            pcx, pcy, pw, ph, pcf = (go(off + j) for j in range(5))
            tcx, tcy, tw, th = (gt(off + j) for j in range(4))
            p = _corners(pcx, pcy, pw, ph)
            t = _corners(tcx, tcy, tw, th)
            inter = _iou16(p, t)
            iou = inter / (pw * ph + tw * th - inter)
            dconf = pcf - iou
            dx = pcx - tcx
            dy = pcy - tcy
            xy = dx * dx + dy * dy
            wh = (pw + tw - jnp.float32(2.0) * _sqrt16(pw * tw)
                  + ph + th - jnp.float32(2.0) * _sqrt16(ph * th))
            term = dconf * dconf + jnp.float32(LAMBDA_COORD) * (xy + wh)
            return iou, pcf, term

        iou1, pcf1, term1 = box_term(0)
        iou2, pcf2, term2 = box_term(5)
        sel2 = iou2 > iou1
        half = jnp.float32(LAMBDA_NOOBJ)
        row = cls + jnp.where(sel2,
                              term2 + half * pcf1 * pcf1,
                              term1 + half * pcf2 * pcf2)
        return acc + jnp.where(t4 > jnp.float32(0.0), row,
                               jnp.float32(0.0))

      return group_body

    pend = {}

    def start(chunk, slot):
        base = (wid * ROWS_PER_W + chunk * CHUNK_ROWS) * COLS
        pend[slot] = (
            pltpu.async_copy(o_hbm.at[pl.ds(base, CHUNK_W)],
                             bufs[slot][0], sems[slot]),
            pltpu.async_copy(t_hbm.at[pl.ds(base, CHUNK_W)],
                             bufs[slot][1], sems[slot]),
        )

    acc = jnp.zeros((16,), jnp.float32)
    start(0, 0)
    for chunk in range(N_CHUNKS):
        slot = chunk % 2
        if chunk + 1 < N_CHUNKS:
            start(chunk + 1, (chunk + 1) % 2)
        for cp in pend[slot]:
            cp.wait()
        acc = plsc.parallel_loop(0, GROUPS, 1, unroll=2, carry=acc)(
            make_group_body(*bufs[slot]))

    vstage[...] = acc
    pltpu.sync_copy(vstage, out_hbm.at[wid])


@jax.jit
def _sc_partials(o_flat, t_flat):
    mesh = plsc.VectorSubcoreMesh(
        core_axis_name="c", subcore_axis_name="s",
        num_cores=NC, num_subcores=NS)
    return pl.kernel(
        _loss_body,
        out_type=jax.ShapeDtypeStruct((NW, 16), jnp.float32),
        mesh=mesh,
        scratch_types=[
            pltpu.VMEM((CHUNK_W,), jnp.float32),
            pltpu.VMEM((CHUNK_W,), jnp.float32),
            pltpu.VMEM((CHUNK_W,), jnp.float32),
            pltpu.VMEM((CHUNK_W,), jnp.float32),
            pltpu.VMEM((16,), jnp.float32),
            pltpu.SemaphoreType.DMA,
            pltpu.SemaphoreType.DMA,
        ],
        compiler_params=pltpu.CompilerParams(needs_layout_passes=False),
    )(o_flat, t_flat)


def kernel(output, target):
    part = _sc_partials(output.reshape(-1), target.reshape(-1))
    return jnp.sum(part) / jnp.float32(BATCH)


# speedup vs baseline: 2.7352x; 2.7352x over previous
import numpy as np

import jax
import jax.numpy as jnp
from jax import lax
from jax.experimental import pallas as pl
from jax.experimental.pallas import tpu as pltpu
from jax.experimental.pallas import tpu_sc as plsc

GRID_NUM = 7.0
LAMBDA_COORD = 5.0
LAMBDA_NOOBJ = 0.5

BATCH = 2048
COLS = 26
NC, NS = 2, 16
NW = NC * NS
BW = BATCH // NW
NBG = BW // 16

_RSQRT_MAGIC = np.int32(0x5F3759DF)


def _sqrt16(x):
    xc = jnp.maximum(x, jnp.float32(1e-12))
    i = lax.bitcast_convert_type(xc, jnp.int32)
    i = _RSQRT_MAGIC - lax.shift_right_arithmetic(i, 1)
    y = lax.bitcast_convert_type(i, jnp.float32)
    half = jnp.float32(0.5) * xc
    for _ in range(2):
        y = y * (jnp.float32(1.5) - half * y * y)
    return xc * y


def _corners(cx, cy, w, h):
    gx = cx * jnp.float32(1.0 / GRID_NUM)
    gy = cy * jnp.float32(1.0 / GRID_NUM)
    hw = jnp.float32(0.5) * w
    hh = jnp.float32(0.5) * h
    return gx - hw, gy - hh, gx + hw, gy + hh


def _intersection16(p, t):
    px0, py0, px1, py1 = p
    tx0, ty0, tx1, ty1 = t
    ltx = jnp.maximum(px0, tx0)
    lty = jnp.maximum(py0, ty0)
    rbx = jnp.minimum(px1, tx1)
    rby = jnp.minimum(py1, ty1)
    iw = jnp.maximum(rbx - ltx, jnp.float32(0.0))
    ih = jnp.maximum(rby - lty, jnp.float32(0.0))
    return iw * ih


def _group(ob, tb, j, boff, acc):
    def go(col):
        return ob.at[col, j][pl.ds(boff, 16)]

    def gt(col):
        return tb.at[col, j][pl.ds(boff, 16)]

    cls = jnp.zeros((16,), jnp.float32)
    for col in range(10, 26):
        d = go(col) - gt(col)
        cls = cls + d * d
    t4 = gt(4)

    def box_term(off):
        pcx, pcy, pw, ph, pcf = (go(off + k) for k in range(5))
        tcx, tcy, tw, th = (gt(off + k) for k in range(4))
        inter = _intersection16(_corners(pcx, pcy, pw, ph),
                                _corners(tcx, tcy, tw, th))
        iou = inter / (pw * ph + tw * th - inter)
        dconf = pcf - iou
        dx = pcx - tcx
        dy = pcy - tcy
        xy = dx * dx + dy * dy
        wh = (pw + tw - jnp.float32(2.0) * _sqrt16(pw * tw)
              + ph + th - jnp.float32(2.0) * _sqrt16(ph * th))
        term = dconf * dconf + jnp.float32(LAMBDA_COORD) * (xy + wh)
        return iou, pcf, term

    iou1, pcf1, term1 = box_term(0)
    iou2, pcf2, term2 = box_term(5)
    sel2 = iou2 > iou1
    half = jnp.float32(LAMBDA_NOOBJ)
    row = cls + jnp.where(sel2,
                          term2 + half * pcf1 * pcf1,
                          term1 + half * pcf2 * pcf2)
    return acc + jnp.where(t4 > jnp.float32(0.0), row, jnp.float32(0.0))


def _loss_body(o_hbm, t_hbm, out_hbm, ob, tb, vstage):
    c = lax.axis_index("c")
    s = lax.axis_index("s")
    wid = s * NC + c
    b0 = pl.multiple_of((wid // 2) * 128, 128)
    hoff = (wid % 2) * BW

    def chunk_body(i, acc):
        pltpu.sync_copy(o_hbm.at[i, :, :, pl.ds(b0, 128)], ob)
        pltpu.sync_copy(t_hbm.at[i, :, :, pl.ds(b0, 128)], tb)

        def j_body(j, acc):
            for bg in range(NBG):
                boff = pl.multiple_of(hoff + bg * 16, 16)
                acc = _group(ob, tb, j, boff, acc)
            return acc

        return lax.fori_loop(0, 7, j_body, acc)

    acc = lax.fori_loop(0, 7, chunk_body, jnp.zeros((16,), jnp.float32))
    vstage[...] = acc
    pltpu.sync_copy(vstage, out_hbm.at[wid])


@jax.jit
def _sc_partials(o_t, t_t):
    mesh = plsc.VectorSubcoreMesh(
        core_axis_name="c", subcore_axis_name="s",
        num_cores=NC, num_subcores=NS)
    return pl.kernel(
        _loss_body,
        out_type=jax.ShapeDtypeStruct((NW, 16), jnp.float32),
        mesh=mesh,
        scratch_types=[
            pltpu.VMEM((COLS, 7, 128), jnp.float32),
            pltpu.VMEM((COLS, 7, 128), jnp.float32),
            pltpu.VMEM((16,), jnp.float32),
        ],
        compiler_params=pltpu.CompilerParams(
            needs_layout_passes=False,
            use_tc_tiling_on_sc=True,
        ),
    )(o_t, t_t)


def kernel(output, target):
    o_t = jnp.transpose(output, (1, 3, 2, 0))
    t_t = jnp.transpose(target, (1, 3, 2, 0))
    part = _sc_partials(o_t, t_t)
    return jnp.sum(part) / jnp.float32(BATCH)


# speedup vs baseline: 3.2745x; 1.1972x over previous
import numpy as np

import jax
import jax.numpy as jnp
from jax import lax
from jax.experimental import pallas as pl
from jax.experimental.pallas import tpu as pltpu
from jax.experimental.pallas import tpu_sc as plsc

GRID_NUM = 7.0
LAMBDA_COORD = 5.0
LAMBDA_NOOBJ = 0.5

BATCH = 2048
COLS = 26
NC, NS = 2, 16
NW = NC * NS
BW = BATCH // NW
NBG = BW // 16

_RSQRT_MAGIC = np.int32(0x5F3759DF)


def _sqrt16(x):
    xc = jnp.maximum(x, jnp.float32(1e-12))
    i = lax.bitcast_convert_type(xc, jnp.int32)
    i = _RSQRT_MAGIC - lax.shift_right_arithmetic(i, 1)
    y = lax.bitcast_convert_type(i, jnp.float32)
    half = jnp.float32(0.5) * xc
    for _ in range(2):
        y = y * (jnp.float32(1.5) - half * y * y)
    return xc * y


def _corners(cx, cy, w, h):
    gx = cx * jnp.float32(1.0 / GRID_NUM)
    gy = cy * jnp.float32(1.0 / GRID_NUM)
    hw = jnp.float32(0.5) * w
    hh = jnp.float32(0.5) * h
    return gx - hw, gy - hh, gx + hw, gy + hh


def _intersection16(p, t):
    px0, py0, px1, py1 = p
    tx0, ty0, tx1, ty1 = t
    ltx = jnp.maximum(px0, tx0)
    lty = jnp.maximum(py0, ty0)
    rbx = jnp.minimum(px1, tx1)
    rby = jnp.minimum(py1, ty1)
    iw = jnp.maximum(rbx - ltx, jnp.float32(0.0))
    ih = jnp.maximum(rby - lty, jnp.float32(0.0))
    return iw * ih


def _group(ob, tb, j, boff, acc):
    def go(col):
        return ob.at[col, j][pl.ds(boff, 16)]

    def gt(col):
        return tb.at[col, j][pl.ds(boff, 16)]

    cls = jnp.zeros((16,), jnp.float32)
    for col in range(10, 26):
        d = go(col) - gt(col)
        cls = cls + d * d
    t4 = gt(4)

    def box_term(off):
        pcx, pcy, pw, ph, pcf = (go(off + k) for k in range(5))
        tcx, tcy, tw, th = (gt(off + k) for k in range(4))
        inter = _intersection16(_corners(pcx, pcy, pw, ph),
                                _corners(tcx, tcy, tw, th))
        iou = inter / (pw * ph + tw * th - inter)
        dconf = pcf - iou
        dx = pcx - tcx
        dy = pcy - tcy
        xy = dx * dx + dy * dy
        wh = (pw + tw - jnp.float32(2.0) * _sqrt16(pw * tw)
              + ph + th - jnp.float32(2.0) * _sqrt16(ph * th))
        term = dconf * dconf + jnp.float32(LAMBDA_COORD) * (xy + wh)
        return iou, pcf, term

    iou1, pcf1, term1 = box_term(0)
    iou2, pcf2, term2 = box_term(5)
    sel2 = iou2 > iou1
    half = jnp.float32(LAMBDA_NOOBJ)
    row = cls + jnp.where(sel2,
                          term2 + half * pcf1 * pcf1,
                          term1 + half * pcf2 * pcf2)
    return acc + jnp.where(t4 > jnp.float32(0.0), row, jnp.float32(0.0))


def _loss_body(o_hbm, t_hbm, out_hbm, ob0, tb0, ob1, tb1, vstage,
               sem0, sem1):
    c = lax.axis_index("c")
    s = lax.axis_index("s")
    wid = s * NC + c
    b0 = pl.multiple_of((wid // 2) * 128, 128)
    hoff = (wid % 2) * BW
    bufs = ((ob0, tb0), (ob1, tb1))
    sems = (sem0, sem1)
    pend = {}

    def start(i, slot):
        pend[slot] = (
            pltpu.async_copy(o_hbm.at[i, :, :, pl.ds(b0, 128)],
                             bufs[slot][0], sems[slot]),
            pltpu.async_copy(t_hbm.at[i, :, :, pl.ds(b0, 128)],
                             bufs[slot][1], sems[slot]),
        )

    acc = jnp.zeros((16,), jnp.float32)
    start(0, 0)
    for i in range(7):
        slot = i % 2
        if i + 1 < 7:
            start(i + 1, (i + 1) % 2)
        for cp in pend[slot]:
            cp.wait()
        ob, tb = bufs[slot]

        def j_body(j, acc, ob=ob, tb=tb):
            for bg in range(NBG):
                boff = pl.multiple_of(hoff + bg * 16, 16)
                acc = _group(ob, tb, j, boff, acc)
            return acc

        acc = lax.fori_loop(0, 7, j_body, acc)

    vstage[...] = acc
    pltpu.sync_copy(vstage, out_hbm.at[wid])


@jax.jit
def _sc_partials(o_t, t_t):
    mesh = plsc.VectorSubcoreMesh(
        core_axis_name="c", subcore_axis_name="s",
        num_cores=NC, num_subcores=NS)
    return pl.kernel(
        _loss_body,
        out_type=jax.ShapeDtypeStruct((NW, 16), jnp.float32),
        mesh=mesh,
        scratch_types=[
            pltpu.VMEM((COLS, 7, 128), jnp.float32),
            pltpu.VMEM((COLS, 7, 128), jnp.float32),
            pltpu.VMEM((COLS, 7, 128), jnp.float32),
            pltpu.VMEM((COLS, 7, 128), jnp.float32),
            pltpu.VMEM((16,), jnp.float32),
            pltpu.SemaphoreType.DMA,
            pltpu.SemaphoreType.DMA,
        ],
        compiler_params=pltpu.CompilerParams(
            needs_layout_passes=False,
            use_tc_tiling_on_sc=True,
        ),
    )(o_t, t_t)


def kernel(output, target):
    o_t = jnp.transpose(output, (1, 3, 2, 0))
    t_t = jnp.transpose(target, (1, 3, 2, 0))
    part = _sc_partials(o_t, t_t)
    return jnp.sum(part) / jnp.float32(BATCH)
